# baseline (speedup 1.0000x reference)
import functools

import jax
import jax.numpy as jnp
from jax import lax
from jax.experimental import pallas as pl
from jax.experimental.pallas import tpu as pltpu
from jax.experimental.pallas import tpu_sc as plsc

N = 10000
D = 128
H = D // 2
E_RAW = 320000
E_VALID = E_RAW + N
NUM_TILES = 16
BLK_E = 128
EROWS = 2688
EE = EROWS * BLK_E
SLAB = N // NUM_TILES
SLABP = 632
PACK = 16384
CAPH = 11776
CAPH_BLKS = CAPH // BLK_E
WB = 80
N_CHUNKS = N // WB
BROWS = 16
HROWS = EROWS // 2
BSTEPS = HROWS // BROWS


def _mesh():
    return plsc.VectorSubcoreMesh(core_axis_name="c", subcore_axis_name="s",
                                  num_cores=2, num_subcores=NUM_TILES)


_SC_PARAMS = pltpu.CompilerParams(needs_layout_passes=False,
                                  use_tc_tiling_on_sc=False)


def _dense_body(xl_ref, xr_ref, w_ref, asv_ref, adv_ref,
                hl_ref, hr_ref, as_ref, ad_ref):
    h = jnp.dot(xl_ref[...], w_ref[:H, :], preferred_element_type=jnp.float32)
    h = h + jnp.dot(xr_ref[...], w_ref[H:, :], preferred_element_type=jnp.float32)
    hl_ref[...] = h[:, :H]
    hr_ref[...] = h[:, H:]
    as_ref[...] = jnp.dot(h, asv_ref[...], preferred_element_type=jnp.float32)
    ad_ref[...] = jnp.dot(h, adv_ref[...], preferred_element_type=jnp.float32)


_DENSE_BLK = 1000


@jax.jit
def _dense(xl, xr, w, a_src, a_dst):
    grid = (N // _DENSE_BLK,)
    return pl.pallas_call(
        _dense_body,
        grid=grid,
        in_specs=[
            pl.BlockSpec((_DENSE_BLK, H), lambda i: (i, 0)),
            pl.BlockSpec((_DENSE_BLK, H), lambda i: (i, 0)),
            pl.BlockSpec((D, D), lambda i: (0, 0)),
            pl.BlockSpec((D, 1), lambda i: (0, 0)),
            pl.BlockSpec((D, 1), lambda i: (0, 0)),
        ],
        out_specs=[
            pl.BlockSpec((_DENSE_BLK, H), lambda i: (i, 0)),
            pl.BlockSpec((_DENSE_BLK, H), lambda i: (i, 0)),
            pl.BlockSpec((_DENSE_BLK, 1), lambda i: (i, 0)),
            pl.BlockSpec((_DENSE_BLK, 1), lambda i: (i, 0)),
        ],
        out_shape=[
            jax.ShapeDtypeStruct((N, H), jnp.float32),
            jax.ShapeDtypeStruct((N, H), jnp.float32),
            jax.ShapeDtypeStruct((N, 1), jnp.float32),
            jax.ShapeDtypeStruct((N, 1), jnp.float32),
        ],
    )(xl, xr, w, a_src, a_dst)


def _build_body(srcH, dstH, elist, counts, srcC, dstC, elist_v, cbuf,
                ss0, ss1, sd0, sd1):
    c = lax.axis_index("c")
    s = lax.axis_index("s")
    sss = (ss0, ss1)
    sds = (sd0, sd1)
    base = s * SLAB
    rbase = c * HROWS

    def _issue(step, buf):
        r0 = pl.multiple_of(rbase + step * BROWS, 8)
        pltpu.async_copy(srcH.at[pl.ds(r0, BROWS)], srcC.at[buf], sss[buf])
        pltpu.async_copy(dstH.at[pl.ds(r0, BROWS)], dstC.at[buf], sds[buf])

    def _proc(step, buf, cnt):
        r0 = pl.multiple_of(rbase + step * BROWS, 8)
        pltpu.make_async_copy(srcH.at[pl.ds(r0, BROWS)], srcC.at[buf],
                              sss[buf]).wait()
        pltpu.make_async_copy(dstH.at[pl.ds(r0, BROWS)], dstC.at[buf],
                              sds[buf]).wait()
        for r in range(BROWS):
            for g in range(BLK_E // 16):
                sl = pl.ds(g * 16, 16)
                sv = srcC[buf, r, sl]
                dv = dstC[buf, r, sl]
                gid = (rbase + step * BROWS + r) * BLK_E + g * 16 \
                    + lax.iota(jnp.int32, 16)
                m = (gid < E_VALID) & (sv >= base) & (sv < base + SLAB)
                pk = sv * PACK + dv
                cc = jnp.minimum(cnt, CAPH - 16)
                plsc.store_compressed(elist_v.at[pl.ds(cc, 16)], pk, mask=m)
                pc = plsc.all_reduce_population_count(m)
                cnt = cnt + pc[0]
        return cnt

    _issue(0, 0)

    def _step(i, cnt):
        j0 = 2 * i
        _issue(j0 + 1, 1)
        cnt = _proc(j0, 0, cnt)
        @pl.when(j0 + 2 < BSTEPS)
        def _():
            _issue(j0 + 2, 0)
        cnt = _proc(j0 + 1, 1, cnt)
        return cnt
    cnt = lax.fori_loop(0, BSTEPS // 2, _step, jnp.int32(0))

    row = c * NUM_TILES + s
    pltpu.sync_copy(elist_v, elist.at[row])
    cbuf[pl.ds(0, 16)] = jnp.full((16,), cnt, jnp.int32)
    pltpu.sync_copy(cbuf, counts.at[row])


_sc_build = pl.kernel(
    _build_body,
    out_type=(jax.ShapeDtypeStruct((2 * NUM_TILES, CAPH), jnp.int32),
              jax.ShapeDtypeStruct((2 * NUM_TILES, 16), jnp.int32)),
    mesh=_mesh(),
    compiler_params=_SC_PARAMS,
    scratch_types=[
        pltpu.VMEM((2, BROWS, BLK_E), jnp.int32),
        pltpu.VMEM((2, BROWS, BLK_E), jnp.int32),
        pltpu.VMEM((CAPH,), jnp.int32),
        pltpu.VMEM((16,), jnp.int32),
        pltpu.SemaphoreType.DMA,
        pltpu.SemaphoreType.DMA,
        pltpu.SemaphoreType.DMA,
        pltpu.SemaphoreType.DMA,
    ],
)


def _sc_body(apply_relu, hL, hR, asrc, adst, elist, counts, bias, out,
             hslab, elist_v, asl, adst_v, rows, exb, dstb, biasv, cstage,
             acc_sh, den_sh):
    c = lax.axis_index("c")
    s = lax.axis_index("s")
    zero16 = jnp.zeros((16,), jnp.float32)
    wb = rows.at[pl.ds(0, WB)]
    wbd = exb.at[0, pl.ds(0, WB)]

    def _zrow(r, carry):
        for kk in range(H // 16):
            wb[r, pl.ds(kk * 16, 16)] = zero16
        return carry
    lax.fori_loop(0, WB, _zrow, 0)
    for kk in range(WB // 16):
        wbd[pl.ds(kk * 16, 16)] = zero16

    def _zchunk(i, carry):
        k = s + 16 * i
        @pl.when(k < N_CHUNKS)
        def _():
            pltpu.sync_copy(wb, acc_sh.at[pl.ds(k * WB, WB)])
            pltpu.sync_copy(wbd, den_sh.at[pl.ds(k * WB, WB)])
        return carry
    lax.fori_loop(0, (N_CHUNKS + 15) // 16, _zchunk, 0)

    base = s * SLAB
    aligned = pl.multiple_of((base // 8) * 8, 8)
    off = base - aligned

    @pl.when(c == 0)
    def _():
        pltpu.sync_copy(hL.at[pl.ds(aligned, SLABP)], hslab)
    @pl.when(c == 1)
    def _():
        pltpu.sync_copy(hR.at[pl.ds(aligned, SLABP)], hslab)
    pltpu.sync_copy(asrc.at[pl.ds(aligned, SLABP)], asl)
    pltpu.sync_copy(adst, adst_v)
    pltpu.sync_copy(elist.at[s], elist_v.at[0])
    pltpu.sync_copy(elist.at[NUM_TILES + s], elist_v.at[1])
    pltpu.sync_copy(counts.at[s], cstage.at[0])
    pltpu.sync_copy(counts.at[NUM_TILES + s], cstage.at[1])
    pltpu.sync_copy(bias.at[c], biasv)

    plsc.subcore_barrier()

    evecs = [g * 16 + lax.iota(jnp.int32, 16) for g in range(BLK_E // 16)]

    def _seg(seg):
        cntv = cstage[seg, pl.ds(0, 16)]
        ev = elist_v.at[seg]

        def _blk(b, carry):
            b0 = pl.multiple_of(b * BLK_E, 8)
            vals = []
            for g in range(BLK_E // 16):
                sl = pl.ds(g * 16, 16)
                pk = ev[pl.ds(b0 + g * 16, 16)]
                ids = b * BLK_E + g * 16 + lax.iota(jnp.int32, 16)
                m = ids < cntv
                lv = lax.shift_right_logical(pk, 14)
                dv = pk - lv * PACK
                lvl = jnp.where(m, lv - aligned, 0)
                dv = jnp.where(m, dv, 0)
                a = plsc.load_gather(asl, [lvl]) + plsc.load_gather(adst_v, [dv])
                a = jnp.where(a >= 0.0, a, 0.2 * a)
                ex = jnp.exp(a)
                ex = jnp.where(m, ex, 0.0)
                exb[0, sl] = ex
                dstb[0, sl] = dv
                vals.append((lvl, ex))
            def _feat(f, cc):
                fv = jnp.full((16,), f, jnp.int32)
                for g in range(BLK_E // 16):
                    lvl, ex = vals[g]
                    hv = plsc.load_gather(hslab, [lvl, fv])
                    plsc.store_scatter(rows, [evecs[g], fv], hv * ex)
                return cc
            lax.fori_loop(0, H, _feat, 0)
            pltpu.sync_copy(exb.at[0], den_sh.at[dstb.at[0]], add=True)
            pltpu.sync_copy(rows, acc_sh.at[dstb.at[0]], add=True)
            return carry
        lax.fori_loop(0, CAPH_BLKS, _blk, 0)

    _seg(0)
    _seg(1)

    plsc.subcore_barrier()

    def _wb(i, carry):
        k = s + 16 * i
        @pl.when(k < N_CHUNKS)
        def _():
            r0 = k * WB
            pltpu.sync_copy(acc_sh.at[pl.ds(r0, WB)], wb)
            pltpu.sync_copy(den_sh.at[pl.ds(r0, WB)], wbd)
            def _row(r, cc):
                d = plsc.load_gather(wbd, [jnp.full((16,), r, jnp.int32)]) + 1e-16
                for kk in range(H // 16):
                    slk = pl.ds(kk * 16, 16)
                    v = wb[r, slk] / d + biasv[slk]
                    if apply_relu:
                        v = jnp.maximum(v, 0.0)
                    wb[r, slk] = v
                return cc
            lax.fori_loop(0, WB, _row, 0)
            pltpu.sync_copy(wb, out.at[c, pl.ds(r0, WB)])
        return carry
    lax.fori_loop(0, (N_CHUNKS + 15) // 16, _wb, 0)


def _make_sc_layer(apply_relu):
    return pl.kernel(
        functools.partial(_sc_body, apply_relu),
        out_type=jax.ShapeDtypeStruct((2, N, H), jnp.float32),
        mesh=_mesh(),
        compiler_params=_SC_PARAMS,
        scratch_types=[
            pltpu.VMEM((SLABP, H), jnp.float32),
            pltpu.VMEM((2, CAPH), jnp.int32),
            pltpu.VMEM((SLABP,), jnp.float32),
            pltpu.VMEM((N,), jnp.float32),
            pltpu.VMEM((BLK_E, H), jnp.float32),
            pltpu.VMEM((1, BLK_E), jnp.float32),
            pltpu.VMEM((1, BLK_E), jnp.int32),
            pltpu.VMEM((H,), jnp.float32),
            pltpu.VMEM((2, 16), jnp.int32),
            pltpu.VMEM_SHARED((N, H), jnp.float32),
            pltpu.VMEM_SHARED((N,), jnp.float32),
        ],
    )


_sc_layer_relu = _make_sc_layer(True)
_sc_layer_plain = _make_sc_layer(False)


@jax.jit
def kernel(x, edge_index, W1, a_src1, a_dst1, b1, W2, a_src2, a_dst2, b2):
    loops = jnp.arange(N, dtype=jnp.int32)
    src = jnp.concatenate([edge_index[0].astype(jnp.int32), loops])
    dst = jnp.concatenate([edge_index[1].astype(jnp.int32), loops])
    pad = EE - E_VALID
    src = jnp.pad(src, (0, pad)).reshape(EROWS, BLK_E)
    dst = jnp.pad(dst, (0, pad)).reshape(EROWS, BLK_E)

    elist, counts = _sc_build(src, dst)

    h1l, h1r, as1, ad1 = _dense(x[:, :H], x[:, H:], W1,
                                a_src1.reshape(D, 1), a_dst1.reshape(D, 1))
    x2 = _sc_layer_relu(h1l, h1r, as1.reshape(N), ad1.reshape(N),
                        elist, counts, b1.reshape(2, H))

    h2l, h2r, as2, ad2 = _dense(x2[0], x2[1], W2,
                                a_src2.reshape(D, 1), a_dst2.reshape(D, 1))
    out2 = _sc_layer_plain(h2l, h2r, as2.reshape(N), ad2.reshape(N),
                           elist, counts, b2.reshape(2, H))

    return jnp.transpose(out2, (1, 0, 2)).reshape(1, N, D)


# speedup vs baseline: 4.4156x; 4.4156x over previous
import functools

import jax
import jax.numpy as jnp
import numpy as np
from jax import lax
from jax.experimental import pallas as pl
from jax.experimental.pallas import tpu as pltpu
from jax.experimental.pallas import tpu_sc as plsc

N = 10000
D = 128
H = D // 2
E_RAW = 320000
E_VALID = E_RAW + N
NUM_TILES = 16
BLK_E = 128
ROWS_PER_TILE = 168
EE = NUM_TILES * ROWS_PER_TILE * BLK_E
NBUF = 2
WB = 80
N_CHUNKS = N // WB


def _dense_body(xl_ref, xr_ref, w_ref, asv_ref, adv_ref, h_ref, as_ref, ad_ref):
    h = jnp.dot(xl_ref[...], w_ref[:H, :], preferred_element_type=jnp.float32)
    h = h + jnp.dot(xr_ref[...], w_ref[H:, :], preferred_element_type=jnp.float32)
    h_ref[...] = h.astype(jnp.bfloat16)
    as_ref[...] = jnp.dot(h, asv_ref[...], preferred_element_type=jnp.float32)
    ad_ref[...] = jnp.dot(h, adv_ref[...], preferred_element_type=jnp.float32)


_DENSE_BLK = 1000


@jax.jit
def _dense(xl, xr, w, a_src, a_dst):
    grid = (N // _DENSE_BLK,)
    return pl.pallas_call(
        _dense_body,
        grid=grid,
        in_specs=[
            pl.BlockSpec((_DENSE_BLK, H), lambda i: (i, 0)),
            pl.BlockSpec((_DENSE_BLK, H), lambda i: (i, 0)),
            pl.BlockSpec((D, D), lambda i: (0, 0)),
            pl.BlockSpec((D, 1), lambda i: (0, 0)),
            pl.BlockSpec((D, 1), lambda i: (0, 0)),
        ],
        out_specs=[
            pl.BlockSpec((_DENSE_BLK, D), lambda i: (i, 0)),
            pl.BlockSpec((_DENSE_BLK, 1), lambda i: (i, 0)),
            pl.BlockSpec((_DENSE_BLK, 1), lambda i: (i, 0)),
        ],
        out_shape=[
            jax.ShapeDtypeStruct((N, D), jnp.bfloat16),
            jax.ShapeDtypeStruct((N, 1), jnp.float32),
            jax.ShapeDtypeStruct((N, 1), jnp.float32),
        ],
    )(xl, xr, w, a_src, a_dst)


def _sc_body(apply_relu, h2, asrc, adst, srcH, dstH, bias, out,
             asrc_v, adst_v, srcI, dstI, gidx, exb, rows, rows_f, biasv,
             acc_sh, den_sh, sems):
    wb = rows_f.at[pl.ds(0, WB)]
    wbd = exb.at[0, pl.ds(0, WB)]
    c = lax.axis_index("c")
    s = lax.axis_index("s")
    zero16 = jnp.zeros((16,), jnp.float32)

    def _zrow(r, carry):
        for kk in range(H // 16):
            wb[r, pl.ds(kk * 16, 16)] = zero16
        return carry
    lax.fori_loop(0, WB, _zrow, 0)
    for kk in range(WB // 16):
        wbd[pl.ds(kk * 16, 16)] = zero16

    def _zchunk(i, carry):
        k = s + 16 * i
        @pl.when(k < N_CHUNKS)
        def _():
            pltpu.sync_copy(wb, acc_sh.at[pl.ds(k * WB, WB)])
            pltpu.sync_copy(wbd, den_sh.at[pl.ds(k * WB, WB)])
        return carry
    lax.fori_loop(0, (N_CHUNKS + 15) // 16, _zchunk, 0)

    pltpu.sync_copy(asrc, asrc_v)
    pltpu.sync_copy(adst, adst_v)
    pltpu.sync_copy(srcH.at[pl.ds(s * ROWS_PER_TILE, ROWS_PER_TILE)], srcI)
    pltpu.sync_copy(dstH.at[pl.ds(s * ROWS_PER_TILE, ROWS_PER_TILE)], dstI)
    pltpu.sync_copy(bias.at[c], biasv)

    plsc.subcore_barrier()

    def _build(j, buf):
        for g in range(BLK_E // 16):
            sl = pl.ds(g * 16, 16)
            sv = srcI[j, sl]
            dv = dstI[j, sl]
            a = plsc.load_gather(asrc_v, [sv]) + plsc.load_gather(adst_v, [dv])
            a = jnp.where(a >= 0.0, a, 0.2 * a)
            ex = jnp.exp(a)
            base = (s * ROWS_PER_TILE + j) * BLK_E + g * 16
            ids = base + lax.iota(jnp.int32, 16)
            ex = jnp.where(ids < E_VALID, ex, 0.0)
            exb[buf, sl] = ex
            gidx[buf, sl] = sv * 2 + c
        pltpu.async_copy(h2.at[gidx.at[buf]],
                         rows.at[pl.ds(buf * BLK_E, BLK_E)], sems.at[buf])
        pltpu.sync_copy(exb.at[buf], den_sh.at[dstI.at[j]], add=True)

    def _consume(j, buf):
        rv = rows.at[pl.ds(buf * BLK_E, BLK_E)]
        pltpu.make_async_copy(h2.at[gidx.at[buf]], rv, sems.at[buf]).wait()
        for g in range(BLK_E // 16):
            cvec = exb[buf, pl.ds(g * 16, 16)]
            for l in range(16):
                e = buf * BLK_E + g * 16 + l
                ef = g * 16 + l
                cs = cvec[l]
                for kk in range(H // 32):
                    ab = rows[e, pl.ds(kk * 32, 32)]
                    av, bv = plsc.unpack(ab, format=plsc.PackFormat.INTERLEAVED)
                    rows_f[ef, pl.ds(kk * 32, 16)] = av * cs
                    rows_f[ef, pl.ds(kk * 32 + 16, 16)] = bv * cs
        pltpu.sync_copy(rows_f, acc_sh.at[dstI.at[j]], add=True)

    _build(0, 0)

    def _blk2(i, carry):
        j0 = 2 * i
        _build(j0 + 1, 1)
        _consume(j0, 0)
        @pl.when(j0 + 2 < ROWS_PER_TILE)
        def _():
            _build(j0 + 2, 0)
        _consume(j0 + 1, 1)
        return carry
    lax.fori_loop(0, ROWS_PER_TILE // 2, _blk2, 0)

    plsc.subcore_barrier()

    def _wb(i, carry):
        k = s + 16 * i
        @pl.when(k < N_CHUNKS)
        def _():
            r0 = k * WB
            pltpu.sync_copy(acc_sh.at[pl.ds(r0, WB)], wb)
            pltpu.sync_copy(den_sh.at[pl.ds(r0, WB)], wbd)
            def _row(r, cc):
                d = plsc.load_gather(wbd, [jnp.full((16,), r, jnp.int32)]) + 1e-16
                for kk in range(H // 16):
                    slk = pl.ds(kk * 16, 16)
                    v = wb[r, slk] / d + biasv[slk]
                    if apply_relu:
                        v = jnp.maximum(v, 0.0)
                    wb[r, slk] = v
                return cc
            lax.fori_loop(0, WB, _row, 0)
            pltpu.sync_copy(wb, out.at[c, pl.ds(r0, WB)])
        return carry
    lax.fori_loop(0, (N_CHUNKS + 15) // 16, _wb, 0)


def _make_sc_layer(apply_relu):
    mesh = plsc.VectorSubcoreMesh(core_axis_name="c", subcore_axis_name="s",
                                  num_cores=2, num_subcores=NUM_TILES)
    return pl.kernel(
        functools.partial(_sc_body, apply_relu),
        out_type=jax.ShapeDtypeStruct((2, N, H), jnp.float32),
        mesh=mesh,
        compiler_params=pltpu.CompilerParams(needs_layout_passes=False,
                                             use_tc_tiling_on_sc=False),
        scratch_types=[
            pltpu.VMEM((N,), jnp.float32),
            pltpu.VMEM((N,), jnp.float32),
            pltpu.VMEM((ROWS_PER_TILE, BLK_E), jnp.int32),
            pltpu.VMEM((ROWS_PER_TILE, BLK_E), jnp.int32),
            pltpu.VMEM((NBUF, BLK_E), jnp.int32),
            pltpu.VMEM((NBUF, BLK_E), jnp.float32),
            pltpu.VMEM((NBUF * BLK_E, H), jnp.bfloat16),
            pltpu.VMEM((BLK_E, H), jnp.float32),
            pltpu.VMEM((H,), jnp.float32),
            pltpu.VMEM_SHARED((N, H), jnp.float32),
            pltpu.VMEM_SHARED((N,), jnp.float32),
            pltpu.SemaphoreType.DMA((NBUF,)),
        ],
    )


_sc_layer_relu = _make_sc_layer(True)
_sc_layer_plain = _make_sc_layer(False)


def _perm_half():
    q = [0] * H
    for kk in range(2):
        for j in range(16):
            q[32 * kk + 2 * j] = 32 * kk + j
            q[32 * kk + 2 * j + 1] = 32 * kk + 16 + j
    return q


_QH = _perm_half()
_Q = np.array([c * H + u for c in range(2) for u in _QH], dtype=np.int32)


@jax.jit
def kernel(x, edge_index, W1, a_src1, a_dst1, b1, W2, a_src2, a_dst2, b2):
    W1p, W2p = W1[:, _Q], W2[:, _Q]
    as1p, ad1p = a_src1[_Q], a_dst1[_Q]
    as2p, ad2p = a_src2[_Q], a_dst2[_Q]
    loops = jnp.arange(N, dtype=jnp.int32)
    src = jnp.concatenate([edge_index[0].astype(jnp.int32), loops])
    dst = jnp.concatenate([edge_index[1].astype(jnp.int32), loops])
    pad = EE - E_VALID
    src = jnp.pad(src, (0, pad)).reshape(EE // BLK_E, BLK_E)
    dst = jnp.pad(dst, (0, pad)).reshape(EE // BLK_E, BLK_E)

    h1, as1, ad1 = _dense(x[:, :H], x[:, H:], W1p,
                          as1p.reshape(D, 1), ad1p.reshape(D, 1))
    x2 = _sc_layer_relu(h1.reshape(2 * N, H), as1.reshape(N), ad1.reshape(N),
                        src, dst, b1.reshape(2, H))

    h2, as2, ad2 = _dense(x2[0], x2[1], W2p,
                          as2p.reshape(D, 1), ad2p.reshape(D, 1))
    out2 = _sc_layer_plain(h2.reshape(2 * N, H), as2.reshape(N), ad2.reshape(N),
                           src, dst, b2.reshape(2, H))

    return jnp.transpose(out2, (1, 0, 2)).reshape(1, N, D)
